# baseline (speedup 1.0000x reference)
import functools

import jax
import jax.numpy as jnp
from jax import lax
from jax.experimental import pallas as pl
from jax.experimental.pallas import tpu as pltpu
from jax.experimental.pallas import tpu_sc as plsc

N_NODES = 100000
HID = 20
NSUB = 16
EB = 128
HC = 16
NP = N_NODES + HC
RPT = NP // NSUB
BN = 2000

_mesh = plsc.VectorSubcoreMesh(core_axis_name="c", subcore_axis_name="s")
_CP = pltpu.CompilerParams(use_tc_tiling_on_sc=False)


KCH = 23
NBUF = 6


def _sc_degree(EP):
    cpt = EP // EB // NSUB
    assert cpt % KCH == 0
    groups = cpt // KCH

    @functools.partial(
        pl.kernel,
        out_type=jax.ShapeDtypeStruct((2, NP, HC), jnp.float32),
        mesh=_mesh,
        compiler_params=_CP,
        scratch_types=[
            pltpu.VMEM((KCH, EB), jnp.int32),
            pltpu.VMEM((EB, HC), jnp.float32),
            pltpu.VMEM_SHARED((NP, HC), jnp.float32),
            pltpu.SemaphoreType.DMA,
        ],
    )
    def deg_kernel(s_hbm, r_hbm, ones_hbm, zeros_hbm, deg_out,
                   idx, ones_v, acc, sem):
        c = lax.axis_index("c")
        s = lax.axis_index("s")
        r0 = s * RPT
        pltpu.sync_copy(zeros_hbm, acc.at[pl.ds(r0, RPT)])
        pltpu.sync_copy(ones_hbm, ones_v)
        plsc.subcore_barrier()

        def scan_edges(e_hbm):
            def body(g, carry):
                pltpu.sync_copy(e_hbm.at[pl.ds(s * cpt + g * KCH, KCH)], idx)
                ds = [pltpu.async_copy(ones_v, acc.at[idx.at[j]], sem,
                                       add=True) for j in range(KCH)]
                for d in ds:
                    d.wait()
                return carry
            lax.fori_loop(0, groups, body, 0)

        @pl.when(c == 0)
        def _():
            scan_edges(s_hbm)

        @pl.when(c == 1)
        def _():
            scan_edges(r_hbm)

        plsc.subcore_barrier()
        pltpu.sync_copy(acc.at[pl.ds(r0, RPT)], deg_out.at[c, pl.ds(r0, RPT)])

    return deg_kernel


def _sc_message(EP):
    cpt = EP // EB // NSUB
    assert cpt % (2 * KCH) == 0
    half_groups = cpt // KCH // 2

    @functools.partial(
        pl.kernel,
        out_type=jax.ShapeDtypeStruct((2, NP, HC), jnp.float32),
        mesh=_mesh,
        compiler_params=_CP,
        scratch_types=(
            [pltpu.VMEM((KCH, EB), jnp.int32) for _ in range(4)]
            + [pltpu.VMEM((EB, HC), jnp.float32) for _ in range(NBUF)]
            + [pltpu.VMEM_SHARED((NP, HC), jnp.float32)]
            + [pltpu.SemaphoreType.DMA for _ in range(2 * NBUF + 4)]
        ),
    )
    def msg_kernel(z0_hbm, z1_hbm, s_hbm, r_hbm, zeros_hbm, y_out, *refs):
        sidx = refs[0:2]
        ridx = refs[2:4]
        rows = refs[4:4 + NBUF]
        acc = refs[4 + NBUF]
        gsem = refs[5 + NBUF: 5 + 2 * NBUF]
        ssem = refs[5 + 2 * NBUF: 5 + 3 * NBUF]
        isem = refs[5 + 3 * NBUF:]
        c = lax.axis_index("c")
        s = lax.axis_index("s")
        r0 = s * RPT
        pltpu.sync_copy(zeros_hbm, acc.at[pl.ds(r0, RPT)])
        plsc.subcore_barrier()

        def fire_idx(base, p):
            pltpu.async_copy(s_hbm.at[pl.ds(base, KCH)], sidx[p], isem[p])
            pltpu.async_copy(r_hbm.at[pl.ds(base, KCH)], ridx[p], isem[2 + p])

        def wait_idx(base, p):
            pltpu.make_async_copy(s_hbm.at[pl.ds(base, KCH)], sidx[p],
                                  isem[p]).wait()
            pltpu.make_async_copy(r_hbm.at[pl.ds(base, KCH)], ridx[p],
                                  isem[2 + p]).wait()

        def process(z_hbm, sb, rb):
            dgat = [None] * NBUF
            dsc = [None] * NBUF
            for b in range(NBUF - 1):
                dgat[b] = pltpu.async_copy(z_hbm.at[sb.at[b]],
                                           rows[b], gsem[b])
            for j in range(KCH):
                b = j % NBUF
                dgat[b].wait()
                dsc[b] = pltpu.async_copy(rows[b], acc.at[rb.at[j]],
                                          ssem[b], add=True)
                nxt = j + NBUF - 1
                if nxt < KCH:
                    nb = nxt % NBUF
                    if dsc[nb] is not None:
                        dsc[nb].wait()
                    dgat[nb] = pltpu.async_copy(z_hbm.at[sb.at[nxt]],
                                                rows[nb], gsem[nb])
            for j in range(max(0, KCH - NBUF), KCH):
                dsc[j % NBUF].wait()

        def scan_edges(z_hbm):
            fire_idx(s * cpt, 0)

            def body(t, carry):
                base0 = s * cpt + 2 * t * KCH
                wait_idx(base0, 0)
                fire_idx(base0 + KCH, 1)
                process(z_hbm, sidx[0], ridx[0])
                wait_idx(base0 + KCH, 1)

                @pl.when(t + 1 < half_groups)
                def _():
                    fire_idx(base0 + 2 * KCH, 0)

                process(z_hbm, sidx[1], ridx[1])
                return carry
            lax.fori_loop(0, half_groups, body, 0)

        @pl.when(c == 0)
        def _():
            scan_edges(z0_hbm)

        @pl.when(c == 1)
        def _():
            scan_edges(z1_hbm)

        plsc.subcore_barrier()
        pltpu.sync_copy(acc.at[pl.ds(r0, RPT)], y_out.at[c, pl.ds(r0, RPT)])

    return msg_kernel


def _softmax_relu(h):
    e = jnp.exp(jnp.maximum(h, 0.0))
    return e / jnp.sum(e, axis=-1, keepdims=True)


def _split_cols(z):
    pad = jnp.zeros((z.shape[0], 2 * HC - HID), z.dtype)
    return z[:, :HC], jnp.concatenate([z[:, HC:], pad], axis=-1)


def _tc1a(x, W1, b1):
    grid = (N_NODES // BN,)

    def body(x_ref, w_ref, b_ref, o0_ref, o1_ref):
        h = jnp.dot(x_ref[...], w_ref[...], preferred_element_type=jnp.float32)
        z0, z1 = _split_cols(_softmax_relu(h + b_ref[...]))
        o0_ref[...] = z0
        o1_ref[...] = z1

    return pl.pallas_call(
        body,
        grid=grid,
        in_specs=[
            pl.BlockSpec((BN, x.shape[1]), lambda i: (i, 0)),
            pl.BlockSpec(W1.shape, lambda i: (0, 0)),
            pl.BlockSpec((1, HID), lambda i: (0, 0)),
        ],
        out_specs=[
            pl.BlockSpec((BN, HC), lambda i: (i, 0)),
            pl.BlockSpec((BN, HC), lambda i: (i, 0)),
        ],
        out_shape=[
            jax.ShapeDtypeStruct((N_NODES, HC), jnp.float32),
            jax.ShapeDtypeStruct((N_NODES, HC), jnp.float32),
        ],
    )(x, W1, b1[None])


def _tc1b(z0u, z1u, deg):
    grid = (N_NODES // BN,)

    def body(z0_ref, z1_ref, d_ref, o0_ref, o1_ref):
        scale = lax.rsqrt(jnp.maximum(d_ref[0][:, 0], 1.0))[:, None]
        o0_ref[...] = z0_ref[...] * scale
        o1_ref[...] = z1_ref[...] * scale

    return pl.pallas_call(
        body,
        grid=grid,
        in_specs=[
            pl.BlockSpec((BN, HC), lambda i: (i, 0)),
            pl.BlockSpec((BN, HC), lambda i: (i, 0)),
            pl.BlockSpec((2, BN, HC), lambda i: (0, i, 0)),
        ],
        out_specs=[
            pl.BlockSpec((BN, HC), lambda i: (i, 0)),
            pl.BlockSpec((BN, HC), lambda i: (i, 0)),
        ],
        out_shape=[
            jax.ShapeDtypeStruct((N_NODES, HC), jnp.float32),
            jax.ShapeDtypeStruct((N_NODES, HC), jnp.float32),
        ],
    )(z0u, z1u, deg)


def _recombine(y_ref, d_ref):
    rd = d_ref[1][:, 0]
    h = jnp.concatenate([y_ref[0], y_ref[1][:, : HID - HC]], axis=-1)
    return h * lax.rsqrt(jnp.maximum(rd, 1.0))[:, None]


def _tc_layer2(y1, W2, b2, deg):
    grid = (N_NODES // BN,)

    def body(y_ref, w_ref, b_ref, d_ref, o0_ref, o1_ref):
        h1 = _recombine(y_ref, d_ref)
        h = jnp.dot(h1, w_ref[...], preferred_element_type=jnp.float32)
        z = _softmax_relu(h + b_ref[...])
        z = z * lax.rsqrt(jnp.maximum(d_ref[0][:, 0], 1.0))[:, None]
        z0, z1 = _split_cols(z)
        o0_ref[...] = z0
        o1_ref[...] = z1

    return pl.pallas_call(
        body,
        grid=grid,
        in_specs=[
            pl.BlockSpec((2, BN, HC), lambda i: (0, i, 0)),
            pl.BlockSpec(W2.shape, lambda i: (0, 0)),
            pl.BlockSpec((1, HID), lambda i: (0, 0)),
            pl.BlockSpec((2, BN, HC), lambda i: (0, i, 0)),
        ],
        out_specs=[
            pl.BlockSpec((BN, HC), lambda i: (i, 0)),
            pl.BlockSpec((BN, HC), lambda i: (i, 0)),
        ],
        out_shape=[
            jax.ShapeDtypeStruct((N_NODES, HC), jnp.float32),
            jax.ShapeDtypeStruct((N_NODES, HC), jnp.float32),
        ],
    )(y1, W2, b2[None], deg)


def _tc3a(x, Wmu_x, bmu, Wls_x, bls):
    grid = (N_NODES // BN,)
    Z = Wmu_x.shape[1]

    def body(x_ref, wmx_ref, bm_ref, wlx_ref, bl_ref, xm_ref, xl_ref):
        xb = x_ref[...]
        xm_ref[...] = jnp.dot(xb, wmx_ref[...],
                              preferred_element_type=jnp.float32) + bm_ref[...]
        xl_ref[...] = jnp.dot(xb, wlx_ref[...],
                              preferred_element_type=jnp.float32) + bl_ref[...]

    return pl.pallas_call(
        body,
        grid=grid,
        in_specs=[
            pl.BlockSpec((BN, x.shape[1]), lambda i: (i, 0)),
            pl.BlockSpec(Wmu_x.shape, lambda i: (0, 0)),
            pl.BlockSpec((1, Z), lambda i: (0, 0)),
            pl.BlockSpec(Wls_x.shape, lambda i: (0, 0)),
            pl.BlockSpec((1, Z), lambda i: (0, 0)),
        ],
        out_specs=[
            pl.BlockSpec((BN, Z), lambda i: (i, 0)),
            pl.BlockSpec((BN, Z), lambda i: (i, 0)),
        ],
        out_shape=[
            jax.ShapeDtypeStruct((N_NODES, Z), jnp.float32),
            jax.ShapeDtypeStruct((N_NODES, Z), jnp.float32),
        ],
    )(x, Wmu_x, bmu[None], Wls_x, bls[None])


def _tc3b(y2, deg, xm, xl, Wmu_h, Wls_h):
    grid = (N_NODES // BN,)
    Z = Wmu_h.shape[1]

    def body(y_ref, d_ref, xm_ref, xl_ref, wmh_ref, wlh_ref,
             mu_ref, ls_ref):
        h2 = _recombine(y_ref, d_ref)
        mu_ref[...] = jnp.dot(h2, wmh_ref[...],
                              preferred_element_type=jnp.float32) + xm_ref[...]
        ls_ref[...] = jnp.dot(h2, wlh_ref[...],
                              preferred_element_type=jnp.float32) + xl_ref[...]

    return pl.pallas_call(
        body,
        grid=grid,
        in_specs=[
            pl.BlockSpec((2, BN, HC), lambda i: (0, i, 0)),
            pl.BlockSpec((2, BN, HC), lambda i: (0, i, 0)),
            pl.BlockSpec((BN, Z), lambda i: (i, 0)),
            pl.BlockSpec((BN, Z), lambda i: (i, 0)),
            pl.BlockSpec(Wmu_h.shape, lambda i: (0, 0)),
            pl.BlockSpec(Wls_h.shape, lambda i: (0, 0)),
        ],
        out_specs=[
            pl.BlockSpec((BN, Z), lambda i: (i, 0)),
            pl.BlockSpec((BN, Z), lambda i: (i, 0)),
        ],
        out_shape=[
            jax.ShapeDtypeStruct((N_NODES, Z), jnp.float32),
            jax.ShapeDtypeStruct((N_NODES, Z), jnp.float32),
        ],
    )(y2, deg, xm, xl, Wmu_h, Wls_h)


def kernel(x, senders, receivers, W1, b1, W2, b2, Wmu, bmu, Wls, bls):
    E = senders.shape[0]
    n = x.shape[0]
    quantum = NSUB * EB * KCH * 2
    EP = ((E + quantum - 1) // quantum) * quantum
    pad = EP - E
    pad_n = jnp.full((pad,), n, jnp.int32)
    s_deg = jnp.concatenate([senders, pad_n]).reshape(-1, EB)
    r_pad = jnp.concatenate([receivers, pad_n]).reshape(-1, EB)
    s_gat = jnp.concatenate([senders, jnp.zeros((pad,), jnp.int32)]).reshape(-1, EB)

    ones_rows = jnp.ones((EB, HC), jnp.float32)
    zeros_rows = jnp.zeros((RPT, HC), jnp.float32)

    deg = _sc_degree(EP)(s_deg, r_pad, ones_rows, zeros_rows)
    z0u, z1u = _tc1a(x, W1, b1)
    xm, xl = _tc3a(x, Wmu[HID:], bmu, Wls[HID:], bls)
    z0, z1 = _tc1b(z0u, z1u, deg)
    y1 = _sc_message(EP)(z0, z1, s_gat, r_pad, zeros_rows)
    z20, z21 = _tc_layer2(y1, W2, b2, deg)
    y2 = _sc_message(EP)(z20, z21, s_gat, r_pad, zeros_rows)
    mu, logsig2 = _tc3b(y2, deg, xm, xl, Wmu[:HID], Wls[:HID])
    return (mu, logsig2)


# speedup vs baseline: 1.0102x; 1.0102x over previous
import functools

import jax
import jax.numpy as jnp
from jax import lax
from jax.experimental import pallas as pl
from jax.experimental.pallas import tpu as pltpu
from jax.experimental.pallas import tpu_sc as plsc

N_NODES = 100000
HID = 20
NSUB = 16
EB = 128
HC = 16
NP = N_NODES + HC
RPT = NP // NSUB
BN = 2000

_mesh = plsc.VectorSubcoreMesh(core_axis_name="c", subcore_axis_name="s")
_CP = pltpu.CompilerParams(use_tc_tiling_on_sc=False)


KCH = 23
NBUF = 6


def _sc_degree(EP):
    cpt = EP // EB // NSUB
    assert cpt % KCH == 0
    groups = cpt // KCH

    @functools.partial(
        pl.kernel,
        out_type=jax.ShapeDtypeStruct((2, NP, HC), jnp.float32),
        mesh=_mesh,
        compiler_params=_CP,
        scratch_types=[
            pltpu.VMEM((KCH, EB), jnp.int32),
            pltpu.VMEM((EB, HC), jnp.float32),
            pltpu.VMEM_SHARED((NP, HC), jnp.float32),
            pltpu.SemaphoreType.DMA,
        ],
    )
    def deg_kernel(s_hbm, r_hbm, ones_hbm, zeros_hbm, deg_out,
                   idx, ones_v, acc, sem):
        c = lax.axis_index("c")
        s = lax.axis_index("s")
        r0 = s * RPT
        pltpu.sync_copy(zeros_hbm, acc.at[pl.ds(r0, RPT)])
        pltpu.sync_copy(ones_hbm, ones_v)
        plsc.subcore_barrier()

        def scan_edges(e_hbm):
            def body(g, carry):
                pltpu.sync_copy(e_hbm.at[pl.ds(s * cpt + g * KCH, KCH)], idx)
                ds = [pltpu.async_copy(ones_v, acc.at[idx.at[j]], sem,
                                       add=True) for j in range(KCH)]
                for d in ds:
                    d.wait()
                return carry
            lax.fori_loop(0, groups, body, 0)

        @pl.when(c == 0)
        def _():
            scan_edges(s_hbm)

        @pl.when(c == 1)
        def _():
            scan_edges(r_hbm)

        plsc.subcore_barrier()
        pltpu.sync_copy(acc.at[pl.ds(r0, RPT)], deg_out.at[c, pl.ds(r0, RPT)])

    return deg_kernel


def _sc_message(EP):
    cpt = EP // EB // NSUB
    assert cpt % (2 * KCH) == 0
    half_groups = cpt // KCH // 2

    @functools.partial(
        pl.kernel,
        out_type=jax.ShapeDtypeStruct((2, NP, HC), jnp.float32),
        mesh=_mesh,
        compiler_params=_CP,
        scratch_types=(
            [pltpu.VMEM((KCH, EB), jnp.int32) for _ in range(4)]
            + [pltpu.VMEM((EB, HC), jnp.float32) for _ in range(NBUF)]
            + [pltpu.VMEM_SHARED((NP, HC), jnp.float32)]
            + [pltpu.SemaphoreType.DMA for _ in range(2 * NBUF + 4)]
        ),
    )
    def msg_kernel(z0_hbm, z1_hbm, s_hbm, r_hbm, zeros_hbm, y_out, *refs):
        sidx = refs[0:2]
        ridx = refs[2:4]
        rows = refs[4:4 + NBUF]
        acc = refs[4 + NBUF]
        gsem = refs[5 + NBUF: 5 + 2 * NBUF]
        ssem = refs[5 + 2 * NBUF: 5 + 3 * NBUF]
        isem = refs[5 + 3 * NBUF:]
        c = lax.axis_index("c")
        s = lax.axis_index("s")
        r0 = s * RPT
        pltpu.sync_copy(zeros_hbm, acc.at[pl.ds(r0, RPT)])
        plsc.subcore_barrier()

        def fire_idx(base, p):
            pltpu.async_copy(s_hbm.at[pl.ds(base, KCH)], sidx[p], isem[p])
            pltpu.async_copy(r_hbm.at[pl.ds(base, KCH)], ridx[p], isem[2 + p])

        def wait_idx(base, p):
            pltpu.make_async_copy(s_hbm.at[pl.ds(base, KCH)], sidx[p],
                                  isem[p]).wait()
            pltpu.make_async_copy(r_hbm.at[pl.ds(base, KCH)], ridx[p],
                                  isem[2 + p]).wait()

        def process(z_hbm, sb, rb):
            dgat = [None] * NBUF
            dsc = [None] * NBUF
            for b in range(NBUF - 1):
                dgat[b] = pltpu.async_copy(z_hbm.at[sb.at[b]],
                                           rows[b], gsem[b])
            for j in range(KCH):
                b = j % NBUF
                dgat[b].wait()
                dsc[b] = pltpu.async_copy(rows[b], acc.at[rb.at[j]],
                                          ssem[b], add=True)
                nxt = j + NBUF - 1
                if nxt < KCH:
                    nb = nxt % NBUF
                    if dsc[nb] is not None:
                        dsc[nb].wait()
                    dgat[nb] = pltpu.async_copy(z_hbm.at[sb.at[nxt]],
                                                rows[nb], gsem[nb])
            for j in range(max(0, KCH - NBUF), KCH):
                dsc[j % NBUF].wait()

        def scan_edges(z_hbm):
            fire_idx(s * cpt, 0)

            def body(t, carry):
                base0 = s * cpt + 2 * t * KCH
                wait_idx(base0, 0)
                fire_idx(base0 + KCH, 1)
                process(z_hbm, sidx[0], ridx[0])
                wait_idx(base0 + KCH, 1)

                @pl.when(t + 1 < half_groups)
                def _():
                    fire_idx(base0 + 2 * KCH, 0)

                process(z_hbm, sidx[1], ridx[1])
                return carry
            lax.fori_loop(0, half_groups, body, 0)

        @pl.when(c == 0)
        def _():
            scan_edges(z0_hbm)

        @pl.when(c == 1)
        def _():
            scan_edges(z1_hbm)

        plsc.subcore_barrier()
        pltpu.sync_copy(acc.at[pl.ds(r0, RPT)], y_out.at[c, pl.ds(r0, RPT)])

    return msg_kernel


def _softmax_relu(h):
    e = jnp.exp(jnp.maximum(h, 0.0))
    return e / jnp.sum(e, axis=-1, keepdims=True)


def _split_cols(z):
    pad = jnp.zeros((z.shape[0], 2 * HC - HID), z.dtype)
    return z[:, :HC], jnp.concatenate([z[:, HC:], pad], axis=-1)


def _tc_layer1(x, W1, b1, deg):
    grid = (N_NODES // BN,)

    def body(x_ref, w_ref, b_ref, d_ref, o0_ref, o1_ref):
        h = jnp.dot(x_ref[...], w_ref[...], preferred_element_type=jnp.float32)
        z = _softmax_relu(h + b_ref[...])
        z = z * lax.rsqrt(jnp.maximum(d_ref[0][:, 0], 1.0))[:, None]
        z0, z1 = _split_cols(z)
        o0_ref[...] = z0
        o1_ref[...] = z1

    return pl.pallas_call(
        body,
        grid=grid,
        in_specs=[
            pl.BlockSpec((BN, x.shape[1]), lambda i: (i, 0)),
            pl.BlockSpec(W1.shape, lambda i: (0, 0)),
            pl.BlockSpec((1, HID), lambda i: (0, 0)),
            pl.BlockSpec((2, BN, HC), lambda i: (0, i, 0)),
        ],
        out_specs=[
            pl.BlockSpec((BN, HC), lambda i: (i, 0)),
            pl.BlockSpec((BN, HC), lambda i: (i, 0)),
        ],
        out_shape=[
            jax.ShapeDtypeStruct((N_NODES, HC), jnp.float32),
            jax.ShapeDtypeStruct((N_NODES, HC), jnp.float32),
        ],
    )(x, W1, b1[None], deg)


def _recombine(y_ref, d_ref):
    rd = d_ref[1][:, 0]
    h = jnp.concatenate([y_ref[0], y_ref[1][:, : HID - HC]], axis=-1)
    return h * lax.rsqrt(jnp.maximum(rd, 1.0))[:, None]


def _tc_layer2(y1, W2, b2, deg):
    grid = (N_NODES // BN,)

    def body(y_ref, w_ref, b_ref, d_ref, o0_ref, o1_ref):
        h1 = _recombine(y_ref, d_ref)
        h = jnp.dot(h1, w_ref[...], preferred_element_type=jnp.float32)
        z = _softmax_relu(h + b_ref[...])
        z = z * lax.rsqrt(jnp.maximum(d_ref[0][:, 0], 1.0))[:, None]
        z0, z1 = _split_cols(z)
        o0_ref[...] = z0
        o1_ref[...] = z1

    return pl.pallas_call(
        body,
        grid=grid,
        in_specs=[
            pl.BlockSpec((2, BN, HC), lambda i: (0, i, 0)),
            pl.BlockSpec(W2.shape, lambda i: (0, 0)),
            pl.BlockSpec((1, HID), lambda i: (0, 0)),
            pl.BlockSpec((2, BN, HC), lambda i: (0, i, 0)),
        ],
        out_specs=[
            pl.BlockSpec((BN, HC), lambda i: (i, 0)),
            pl.BlockSpec((BN, HC), lambda i: (i, 0)),
        ],
        out_shape=[
            jax.ShapeDtypeStruct((N_NODES, HC), jnp.float32),
            jax.ShapeDtypeStruct((N_NODES, HC), jnp.float32),
        ],
    )(y1, W2, b2[None], deg)


def _tc3a(x, Wmu_x, bmu, Wls_x, bls):
    grid = (N_NODES // BN,)
    Z = Wmu_x.shape[1]

    def body(x_ref, wmx_ref, bm_ref, wlx_ref, bl_ref, xm_ref, xl_ref):
        xb = x_ref[...]
        xm_ref[...] = jnp.dot(xb, wmx_ref[...],
                              preferred_element_type=jnp.float32) + bm_ref[...]
        xl_ref[...] = jnp.dot(xb, wlx_ref[...],
                              preferred_element_type=jnp.float32) + bl_ref[...]

    return pl.pallas_call(
        body,
        grid=grid,
        in_specs=[
            pl.BlockSpec((BN, x.shape[1]), lambda i: (i, 0)),
            pl.BlockSpec(Wmu_x.shape, lambda i: (0, 0)),
            pl.BlockSpec((1, Z), lambda i: (0, 0)),
            pl.BlockSpec(Wls_x.shape, lambda i: (0, 0)),
            pl.BlockSpec((1, Z), lambda i: (0, 0)),
        ],
        out_specs=[
            pl.BlockSpec((BN, Z), lambda i: (i, 0)),
            pl.BlockSpec((BN, Z), lambda i: (i, 0)),
        ],
        out_shape=[
            jax.ShapeDtypeStruct((N_NODES, Z), jnp.float32),
            jax.ShapeDtypeStruct((N_NODES, Z), jnp.float32),
        ],
    )(x, Wmu_x, bmu[None], Wls_x, bls[None])


def _tc3b(y2, deg, xm, xl, Wmu_h, Wls_h):
    grid = (N_NODES // BN,)
    Z = Wmu_h.shape[1]

    def body(y_ref, d_ref, xm_ref, xl_ref, wmh_ref, wlh_ref,
             mu_ref, ls_ref):
        h2 = _recombine(y_ref, d_ref)
        mu_ref[...] = jnp.dot(h2, wmh_ref[...],
                              preferred_element_type=jnp.float32) + xm_ref[...]
        ls_ref[...] = jnp.dot(h2, wlh_ref[...],
                              preferred_element_type=jnp.float32) + xl_ref[...]

    return pl.pallas_call(
        body,
        grid=grid,
        in_specs=[
            pl.BlockSpec((2, BN, HC), lambda i: (0, i, 0)),
            pl.BlockSpec((2, BN, HC), lambda i: (0, i, 0)),
            pl.BlockSpec((BN, Z), lambda i: (i, 0)),
            pl.BlockSpec((BN, Z), lambda i: (i, 0)),
            pl.BlockSpec(Wmu_h.shape, lambda i: (0, 0)),
            pl.BlockSpec(Wls_h.shape, lambda i: (0, 0)),
        ],
        out_specs=[
            pl.BlockSpec((BN, Z), lambda i: (i, 0)),
            pl.BlockSpec((BN, Z), lambda i: (i, 0)),
        ],
        out_shape=[
            jax.ShapeDtypeStruct((N_NODES, Z), jnp.float32),
            jax.ShapeDtypeStruct((N_NODES, Z), jnp.float32),
        ],
    )(y2, deg, xm, xl, Wmu_h, Wls_h)


def kernel(x, senders, receivers, W1, b1, W2, b2, Wmu, bmu, Wls, bls):
    E = senders.shape[0]
    n = x.shape[0]
    quantum = NSUB * EB * KCH * 2
    EP = ((E + quantum - 1) // quantum) * quantum
    pad = EP - E
    pad_n = jnp.full((pad,), n, jnp.int32)
    s_deg = jnp.concatenate([senders, pad_n]).reshape(-1, EB)
    r_pad = jnp.concatenate([receivers, pad_n]).reshape(-1, EB)
    s_gat = jnp.concatenate([senders, jnp.zeros((pad,), jnp.int32)]).reshape(-1, EB)

    ones_rows = jnp.ones((EB, HC), jnp.float32)
    zeros_rows = jnp.zeros((RPT, HC), jnp.float32)

    deg = _sc_degree(EP)(s_deg, r_pad, ones_rows, zeros_rows)
    z0, z1 = _tc_layer1(x, W1, b1, deg)
    y1 = _sc_message(EP)(z0, z1, s_gat, r_pad, zeros_rows)
    z20, z21 = _tc_layer2(y1, W2, b2, deg)
    y2 = _sc_message(EP)(z20, z21, s_gat, r_pad, zeros_rows)
    mu, logsig2 = _tc3b(y2, deg,
                        *_tc3a(x, Wmu[HID:], bmu, Wls[HID:], bls),
                        Wmu[:HID], Wls[:HID])
    return (mu, logsig2)


# speedup vs baseline: 1.0483x; 1.0377x over previous
import functools

import jax
import jax.numpy as jnp
from jax import lax
from jax.experimental import pallas as pl
from jax.experimental.pallas import tpu as pltpu
from jax.experimental.pallas import tpu_sc as plsc

N_NODES = 100000
HID = 20
NSUB = 16
EB = 128
HC = 16
NP = N_NODES + HC
RPT = NP // NSUB
BN = 2000

_mesh = plsc.VectorSubcoreMesh(core_axis_name="c", subcore_axis_name="s")
_CP = pltpu.CompilerParams(use_tc_tiling_on_sc=False)


KCH = 23
NBUF = 6


def _sc_degree(EP):
    cpt = EP // EB // NSUB
    assert cpt % KCH == 0
    groups = cpt // KCH

    @functools.partial(
        pl.kernel,
        out_type=jax.ShapeDtypeStruct((2, NP, HC), jnp.float32),
        mesh=_mesh,
        compiler_params=_CP,
        scratch_types=[
            pltpu.VMEM((KCH, EB), jnp.int32),
            pltpu.VMEM((EB, HC), jnp.float32),
            pltpu.VMEM_SHARED((NP, HC), jnp.float32),
            pltpu.SemaphoreType.DMA,
        ],
    )
    def deg_kernel(s_hbm, r_hbm, ones_hbm, zeros_hbm, deg_out,
                   idx, ones_v, acc, sem):
        c = lax.axis_index("c")
        s = lax.axis_index("s")
        r0 = s * RPT
        pltpu.sync_copy(zeros_hbm, acc.at[pl.ds(r0, RPT)])
        pltpu.sync_copy(ones_hbm, ones_v)
        plsc.subcore_barrier()

        def scan_edges(e_hbm):
            def body(g, carry):
                pltpu.sync_copy(e_hbm.at[pl.ds(s * cpt + g * KCH, KCH)], idx)
                ds = [pltpu.async_copy(ones_v, acc.at[idx.at[j]], sem,
                                       add=True) for j in range(KCH)]
                for d in ds:
                    d.wait()
                return carry
            lax.fori_loop(0, groups, body, 0)

        @pl.when(c == 0)
        def _():
            scan_edges(s_hbm)

        @pl.when(c == 1)
        def _():
            scan_edges(r_hbm)

        plsc.subcore_barrier()
        pltpu.sync_copy(acc.at[pl.ds(r0, RPT)], deg_out.at[c, pl.ds(r0, RPT)])

    return deg_kernel


def _sc_message(EP):
    cpt = EP // EB // NSUB
    assert cpt % (2 * KCH) == 0
    half_groups = cpt // KCH // 2

    @functools.partial(
        pl.kernel,
        out_type=jax.ShapeDtypeStruct((2, NP, HC), jnp.float32),
        mesh=_mesh,
        compiler_params=_CP,
        scratch_types=(
            [pltpu.VMEM((KCH, EB), jnp.int32) for _ in range(4)]
            + [pltpu.VMEM((EB, HC), jnp.float32) for _ in range(NBUF)]
            + [pltpu.VMEM_SHARED((NP, HC), jnp.float32)]
            + [pltpu.SemaphoreType.DMA for _ in range(2 * NBUF + 4)]
        ),
    )
    def msg_kernel(z0_hbm, z1_hbm, s_hbm, r_hbm, zeros_hbm, y_out, *refs):
        sidx = refs[0:2]
        ridx = refs[2:4]
        rows = refs[4:4 + NBUF]
        acc = refs[4 + NBUF]
        gsem = refs[5 + NBUF: 5 + 2 * NBUF]
        ssem = refs[5 + 2 * NBUF: 5 + 3 * NBUF]
        isem = refs[5 + 3 * NBUF:]
        c = lax.axis_index("c")
        s = lax.axis_index("s")
        r0 = s * RPT
        pltpu.sync_copy(zeros_hbm, acc.at[pl.ds(r0, RPT)])
        plsc.subcore_barrier()

        def fire_idx(base, p):
            pltpu.async_copy(s_hbm.at[pl.ds(base, KCH)], sidx[p], isem[p])
            pltpu.async_copy(r_hbm.at[pl.ds(base, KCH)], ridx[p], isem[2 + p])

        def wait_idx(base, p):
            pltpu.make_async_copy(s_hbm.at[pl.ds(base, KCH)], sidx[p],
                                  isem[p]).wait()
            pltpu.make_async_copy(r_hbm.at[pl.ds(base, KCH)], ridx[p],
                                  isem[2 + p]).wait()

        def process(z_hbm, sb, rb):
            dgat = [None] * NBUF
            dsc = [None] * NBUF
            for b in range(NBUF - 1):
                dgat[b] = pltpu.async_copy(z_hbm.at[sb.at[b]],
                                           rows[b], gsem[b])
            for j in range(KCH):
                b = j % NBUF
                dgat[b].wait()
                dsc[b] = pltpu.async_copy(rows[b], acc.at[rb.at[j]],
                                          ssem[b], add=True)
                nxt = j + NBUF - 1
                if nxt < KCH:
                    nb = nxt % NBUF
                    if dsc[nb] is not None:
                        dsc[nb].wait()
                    dgat[nb] = pltpu.async_copy(z_hbm.at[sb.at[nxt]],
                                                rows[nb], gsem[nb])
            for j in range(max(0, KCH - NBUF), KCH):
                dsc[j % NBUF].wait()

        def scan_edges(z_hbm):
            fire_idx(s * cpt, 0)

            def body(t, carry):
                base0 = s * cpt + 2 * t * KCH
                wait_idx(base0, 0)
                fire_idx(base0 + KCH, 1)
                process(z_hbm, sidx[0], ridx[0])
                wait_idx(base0 + KCH, 1)

                @pl.when(t + 1 < half_groups)
                def _():
                    fire_idx(base0 + 2 * KCH, 0)

                process(z_hbm, sidx[1], ridx[1])
                return carry
            lax.fori_loop(0, half_groups, body, 0)

        @pl.when(c == 0)
        def _():
            scan_edges(z0_hbm)

        @pl.when(c == 1)
        def _():
            scan_edges(z1_hbm)

        plsc.subcore_barrier()
        pltpu.sync_copy(acc.at[pl.ds(r0, RPT)], y_out.at[c, pl.ds(r0, RPT)])

    return msg_kernel


def _softmax_relu(h):
    e = jnp.exp(jnp.maximum(h, 0.0))
    return e / jnp.sum(e, axis=-1, keepdims=True)


def _split_cols(z):
    pad = jnp.zeros((z.shape[0], 2 * HC - HID), z.dtype)
    return z[:, :HC], jnp.concatenate([z[:, HC:], pad], axis=-1)


def _tc_layer1(x, W1, b1, deg):
    grid = (N_NODES // BN,)

    def body(x_ref, w_ref, b_ref, d_ref, o0_ref, o1_ref):
        h = jnp.dot(x_ref[...], w_ref[...], preferred_element_type=jnp.float32)
        z = _softmax_relu(h + b_ref[...])
        z = z * lax.rsqrt(jnp.maximum(d_ref[0][:, 0], 1.0))[:, None]
        z0, z1 = _split_cols(z)
        o0_ref[...] = z0
        o1_ref[...] = z1

    return pl.pallas_call(
        body,
        grid=grid,
        in_specs=[
            pl.BlockSpec((BN, x.shape[1]), lambda i: (i, 0)),
            pl.BlockSpec(W1.shape, lambda i: (0, 0)),
            pl.BlockSpec((1, HID), lambda i: (0, 0)),
            pl.BlockSpec((2, BN, HC), lambda i: (0, i, 0)),
        ],
        out_specs=[
            pl.BlockSpec((BN, HC), lambda i: (i, 0)),
            pl.BlockSpec((BN, HC), lambda i: (i, 0)),
        ],
        out_shape=[
            jax.ShapeDtypeStruct((N_NODES, HC), jnp.float32),
            jax.ShapeDtypeStruct((N_NODES, HC), jnp.float32),
        ],
    )(x, W1, b1[None], deg)


def _recombine(y_ref, d_ref):
    rd = d_ref[1][:, 0]
    h = jnp.concatenate([y_ref[0], y_ref[1][:, : HID - HC]], axis=-1)
    return h * lax.rsqrt(jnp.maximum(rd, 1.0))[:, None]


def _tc_layer2(y1, W2, b2, deg):
    grid = (N_NODES // BN,)

    def body(y_ref, w_ref, b_ref, d_ref, o0_ref, o1_ref):
        h1 = _recombine(y_ref, d_ref)
        h = jnp.dot(h1, w_ref[...], preferred_element_type=jnp.float32)
        z = _softmax_relu(h + b_ref[...])
        z = z * lax.rsqrt(jnp.maximum(d_ref[0][:, 0], 1.0))[:, None]
        z0, z1 = _split_cols(z)
        o0_ref[...] = z0
        o1_ref[...] = z1

    return pl.pallas_call(
        body,
        grid=grid,
        in_specs=[
            pl.BlockSpec((2, BN, HC), lambda i: (0, i, 0)),
            pl.BlockSpec(W2.shape, lambda i: (0, 0)),
            pl.BlockSpec((1, HID), lambda i: (0, 0)),
            pl.BlockSpec((2, BN, HC), lambda i: (0, i, 0)),
        ],
        out_specs=[
            pl.BlockSpec((BN, HC), lambda i: (i, 0)),
            pl.BlockSpec((BN, HC), lambda i: (i, 0)),
        ],
        out_shape=[
            jax.ShapeDtypeStruct((N_NODES, HC), jnp.float32),
            jax.ShapeDtypeStruct((N_NODES, HC), jnp.float32),
        ],
    )(y1, W2, b2[None], deg)


def _tc_heads(y2, deg, x, Wmu_h, Wmu_x, bmu, Wls_h, Wls_x, bls):
    grid = (N_NODES // BN,)
    Z = Wmu_h.shape[1]

    def body(y_ref, d_ref, x_ref, wmh_ref, wmx_ref, bm_ref,
             wlh_ref, wlx_ref, bl_ref, mu_ref, ls_ref):
        h2 = _recombine(y_ref, d_ref)
        xb = x_ref[...]
        mu_ref[...] = (jnp.dot(h2, wmh_ref[...], preferred_element_type=jnp.float32)
                       + jnp.dot(xb, wmx_ref[...], preferred_element_type=jnp.float32)
                       + bm_ref[...])
        ls_ref[...] = (jnp.dot(h2, wlh_ref[...], preferred_element_type=jnp.float32)
                       + jnp.dot(xb, wlx_ref[...], preferred_element_type=jnp.float32)
                       + bl_ref[...])

    return pl.pallas_call(
        body,
        grid=grid,
        in_specs=[
            pl.BlockSpec((2, BN, HC), lambda i: (0, i, 0)),
            pl.BlockSpec((2, BN, HC), lambda i: (0, i, 0)),
            pl.BlockSpec((BN, x.shape[1]), lambda i: (i, 0)),
            pl.BlockSpec(Wmu_h.shape, lambda i: (0, 0)),
            pl.BlockSpec(Wmu_x.shape, lambda i: (0, 0)),
            pl.BlockSpec((1, Z), lambda i: (0, 0)),
            pl.BlockSpec(Wls_h.shape, lambda i: (0, 0)),
            pl.BlockSpec(Wls_x.shape, lambda i: (0, 0)),
            pl.BlockSpec((1, Z), lambda i: (0, 0)),
        ],
        out_specs=[
            pl.BlockSpec((BN, Z), lambda i: (i, 0)),
            pl.BlockSpec((BN, Z), lambda i: (i, 0)),
        ],
        out_shape=[
            jax.ShapeDtypeStruct((N_NODES, Z), jnp.float32),
            jax.ShapeDtypeStruct((N_NODES, Z), jnp.float32),
        ],
    )(y2, deg, x, Wmu_h, Wmu_x, bmu[None], Wls_h, Wls_x, bls[None])


def kernel(x, senders, receivers, W1, b1, W2, b2, Wmu, bmu, Wls, bls):
    E = senders.shape[0]
    n = x.shape[0]
    quantum = NSUB * EB * KCH * 2
    EP = ((E + quantum - 1) // quantum) * quantum
    pad = EP - E
    pad_n = jnp.full((pad,), n, jnp.int32)
    s_deg = jnp.concatenate([senders, pad_n]).reshape(-1, EB)
    r_pad = jnp.concatenate([receivers, pad_n]).reshape(-1, EB)
    s_gat = jnp.concatenate([senders, jnp.zeros((pad,), jnp.int32)]).reshape(-1, EB)

    ones_rows = jnp.ones((EB, HC), jnp.float32)
    zeros_rows = jnp.zeros((RPT, HC), jnp.float32)

    deg = _sc_degree(EP)(s_deg, r_pad, ones_rows, zeros_rows)
    z0, z1 = _tc_layer1(x, W1, b1, deg)
    y1 = _sc_message(EP)(z0, z1, s_gat, r_pad, zeros_rows)
    z20, z21 = _tc_layer2(y1, W2, b2, deg)
    y2 = _sc_message(EP)(z20, z21, s_gat, r_pad, zeros_rows)
    mu, logsig2 = _tc_heads(y2, deg, x, Wmu[:HID], Wmu[HID:], bmu,
                            Wls[:HID], Wls[HID:], bls)
    return (mu, logsig2)


# speedup vs baseline: 1.1041x; 1.0532x over previous
import functools

import jax
import jax.numpy as jnp
from jax import lax
from jax.experimental import pallas as pl
from jax.experimental.pallas import tpu as pltpu
from jax.experimental.pallas import tpu_sc as plsc

N_NODES = 100000
HID = 20
NSUB = 16
EB = 128
HC = 16
NP = N_NODES + HC
RPT = NP // NSUB
BN = 2000

_mesh = plsc.VectorSubcoreMesh(core_axis_name="c", subcore_axis_name="s")
_CP = pltpu.CompilerParams(use_tc_tiling_on_sc=False)


KCH = 23
NBUF = 8


def _sc_degree(EP):
    cpt = EP // EB // NSUB
    assert cpt % (2 * KCH) == 0
    half_groups = cpt // KCH // 2

    @functools.partial(
        pl.kernel,
        out_type=jax.ShapeDtypeStruct((2, NP, HC), jnp.float32),
        mesh=_mesh,
        compiler_params=_CP,
        scratch_types=[
            pltpu.VMEM((KCH, EB), jnp.int32),
            pltpu.VMEM((KCH, EB), jnp.int32),
            pltpu.VMEM((EB, HC), jnp.float32),
            pltpu.VMEM_SHARED((NP, HC), jnp.float32),
            pltpu.SemaphoreType.DMA,
            pltpu.SemaphoreType.DMA,
            pltpu.SemaphoreType.DMA,
        ],
    )
    def deg_kernel(s_hbm, r_hbm, ones_hbm, zeros_hbm, deg_out,
                   idx0, idx1, ones_v, acc, sem, isem0, isem1):
        c = lax.axis_index("c")
        s = lax.axis_index("s")
        r0 = s * RPT
        idx = (idx0, idx1)
        isem = (isem0, isem1)
        pltpu.sync_copy(zeros_hbm, acc.at[pl.ds(r0, RPT)])
        pltpu.sync_copy(ones_hbm, ones_v)
        plsc.subcore_barrier()

        def scan_edges(e_hbm):
            def scatter_block(p):
                ds = [pltpu.async_copy(ones_v, acc.at[idx[p].at[j]], sem,
                                       add=True) for j in range(KCH)]
                for d in ds:
                    d.wait()

            pltpu.async_copy(e_hbm.at[pl.ds(s * cpt, KCH)], idx[0], isem[0])

            def body(t, carry):
                base0 = s * cpt + 2 * t * KCH
                pltpu.make_async_copy(e_hbm.at[pl.ds(base0, KCH)],
                                      idx[0], isem[0]).wait()
                pltpu.async_copy(e_hbm.at[pl.ds(base0 + KCH, KCH)],
                                 idx[1], isem[1])
                scatter_block(0)
                pltpu.make_async_copy(e_hbm.at[pl.ds(base0 + KCH, KCH)],
                                      idx[1], isem[1]).wait()

                @pl.when(t + 1 < half_groups)
                def _():
                    pltpu.async_copy(e_hbm.at[pl.ds(base0 + 2 * KCH, KCH)],
                                     idx[0], isem[0])

                scatter_block(1)
                return carry
            lax.fori_loop(0, half_groups, body, 0)

        @pl.when(c == 0)
        def _():
            scan_edges(s_hbm)

        @pl.when(c == 1)
        def _():
            scan_edges(r_hbm)

        plsc.subcore_barrier()
        pltpu.sync_copy(acc.at[pl.ds(r0, RPT)], deg_out.at[c, pl.ds(r0, RPT)])

    return deg_kernel


def _sc_message(EP):
    cpt = EP // EB // NSUB
    assert cpt % (2 * KCH) == 0
    half_groups = cpt // KCH // 2

    @functools.partial(
        pl.kernel,
        out_type=jax.ShapeDtypeStruct((2, NP, HC), jnp.float32),
        mesh=_mesh,
        compiler_params=_CP,
        scratch_types=(
            [pltpu.VMEM((KCH, EB), jnp.int32) for _ in range(4)]
            + [pltpu.VMEM((EB, HC), jnp.float32) for _ in range(NBUF)]
            + [pltpu.VMEM_SHARED((NP, HC), jnp.float32)]
            + [pltpu.SemaphoreType.DMA for _ in range(2 * NBUF + 4)]
        ),
    )
    def msg_kernel(z0_hbm, z1_hbm, s_hbm, r_hbm, zeros_hbm, y_out, *refs):
        sidx = refs[0:2]
        ridx = refs[2:4]
        rows = refs[4:4 + NBUF]
        acc = refs[4 + NBUF]
        gsem = refs[5 + NBUF: 5 + 2 * NBUF]
        ssem = refs[5 + 2 * NBUF: 5 + 3 * NBUF]
        isem = refs[5 + 3 * NBUF:]
        c = lax.axis_index("c")
        s = lax.axis_index("s")
        r0 = s * RPT
        pltpu.sync_copy(zeros_hbm, acc.at[pl.ds(r0, RPT)])
        plsc.subcore_barrier()

        def fire_idx(base, p):
            pltpu.async_copy(s_hbm.at[pl.ds(base, KCH)], sidx[p], isem[p])
            pltpu.async_copy(r_hbm.at[pl.ds(base, KCH)], ridx[p], isem[2 + p])

        def wait_idx(base, p):
            pltpu.make_async_copy(s_hbm.at[pl.ds(base, KCH)], sidx[p],
                                  isem[p]).wait()
            pltpu.make_async_copy(r_hbm.at[pl.ds(base, KCH)], ridx[p],
                                  isem[2 + p]).wait()

        def process(z_hbm, sb, rb):
            dgat = [None] * NBUF
            dsc = [None] * NBUF
            for b in range(NBUF - 1):
                dgat[b] = pltpu.async_copy(z_hbm.at[sb.at[b]],
                                           rows[b], gsem[b])
            for j in range(KCH):
                b = j % NBUF
                dgat[b].wait()
                dsc[b] = pltpu.async_copy(rows[b], acc.at[rb.at[j]],
                                          ssem[b], add=True)
                nxt = j + NBUF - 1
                if nxt < KCH:
                    nb = nxt % NBUF
                    if dsc[nb] is not None:
                        dsc[nb].wait()
                    dgat[nb] = pltpu.async_copy(z_hbm.at[sb.at[nxt]],
                                                rows[nb], gsem[nb])
            for j in range(max(0, KCH - NBUF), KCH):
                dsc[j % NBUF].wait()

        def scan_edges(z_hbm):
            fire_idx(s * cpt, 0)

            def body(t, carry):
                base0 = s * cpt + 2 * t * KCH
                wait_idx(base0, 0)
                fire_idx(base0 + KCH, 1)
                process(z_hbm, sidx[0], ridx[0])
                wait_idx(base0 + KCH, 1)

                @pl.when(t + 1 < half_groups)
                def _():
                    fire_idx(base0 + 2 * KCH, 0)

                process(z_hbm, sidx[1], ridx[1])
                return carry
            lax.fori_loop(0, half_groups, body, 0)

        @pl.when(c == 0)
        def _():
            scan_edges(z0_hbm)

        @pl.when(c == 1)
        def _():
            scan_edges(z1_hbm)

        plsc.subcore_barrier()
        pltpu.sync_copy(acc.at[pl.ds(r0, RPT)], y_out.at[c, pl.ds(r0, RPT)])

    return msg_kernel


def _softmax_relu(h):
    e = jnp.exp(jnp.maximum(h, 0.0))
    return e / jnp.sum(e, axis=-1, keepdims=True)


def _split_cols(z):
    pad = jnp.zeros((z.shape[0], 2 * HC - HID), z.dtype)
    return z[:, :HC], jnp.concatenate([z[:, HC:], pad], axis=-1)


def _tc_layer1(x, W1, b1, deg):
    grid = (N_NODES // BN,)

    def body(x_ref, w_ref, b_ref, d_ref, o0_ref, o1_ref):
        h = jnp.dot(x_ref[...], w_ref[...], preferred_element_type=jnp.float32)
        z = _softmax_relu(h + b_ref[...])
        z = z * lax.rsqrt(jnp.maximum(d_ref[0][:, 0], 1.0))[:, None]
        z0, z1 = _split_cols(z)
        o0_ref[...] = z0
        o1_ref[...] = z1

    return pl.pallas_call(
        body,
        grid=grid,
        in_specs=[
            pl.BlockSpec((BN, x.shape[1]), lambda i: (i, 0)),
            pl.BlockSpec(W1.shape, lambda i: (0, 0)),
            pl.BlockSpec((1, HID), lambda i: (0, 0)),
            pl.BlockSpec((2, BN, HC), lambda i: (0, i, 0)),
        ],
        out_specs=[
            pl.BlockSpec((BN, HC), lambda i: (i, 0)),
            pl.BlockSpec((BN, HC), lambda i: (i, 0)),
        ],
        out_shape=[
            jax.ShapeDtypeStruct((N_NODES, HC), jnp.float32),
            jax.ShapeDtypeStruct((N_NODES, HC), jnp.float32),
        ],
    )(x, W1, b1[None], deg)


def _recombine(y_ref, d_ref):
    rd = d_ref[1][:, 0]
    h = jnp.concatenate([y_ref[0], y_ref[1][:, : HID - HC]], axis=-1)
    return h * lax.rsqrt(jnp.maximum(rd, 1.0))[:, None]


def _tc_layer2(y1, W2, b2, deg):
    grid = (N_NODES // BN,)

    def body(y_ref, w_ref, b_ref, d_ref, o0_ref, o1_ref):
        h1 = _recombine(y_ref, d_ref)
        h = jnp.dot(h1, w_ref[...], preferred_element_type=jnp.float32)
        z = _softmax_relu(h + b_ref[...])
        z = z * lax.rsqrt(jnp.maximum(d_ref[0][:, 0], 1.0))[:, None]
        z0, z1 = _split_cols(z)
        o0_ref[...] = z0
        o1_ref[...] = z1

    return pl.pallas_call(
        body,
        grid=grid,
        in_specs=[
            pl.BlockSpec((2, BN, HC), lambda i: (0, i, 0)),
            pl.BlockSpec(W2.shape, lambda i: (0, 0)),
            pl.BlockSpec((1, HID), lambda i: (0, 0)),
            pl.BlockSpec((2, BN, HC), lambda i: (0, i, 0)),
        ],
        out_specs=[
            pl.BlockSpec((BN, HC), lambda i: (i, 0)),
            pl.BlockSpec((BN, HC), lambda i: (i, 0)),
        ],
        out_shape=[
            jax.ShapeDtypeStruct((N_NODES, HC), jnp.float32),
            jax.ShapeDtypeStruct((N_NODES, HC), jnp.float32),
        ],
    )(y1, W2, b2[None], deg)


def _tc_heads(y2, deg, x, Wmu_h, Wmu_x, bmu, Wls_h, Wls_x, bls):
    grid = (N_NODES // BN,)
    Z = Wmu_h.shape[1]

    def body(y_ref, d_ref, x_ref, wmh_ref, wmx_ref, bm_ref,
             wlh_ref, wlx_ref, bl_ref, mu_ref, ls_ref):
        h2 = _recombine(y_ref, d_ref)
        xb = x_ref[...]
        mu_ref[...] = (jnp.dot(h2, wmh_ref[...], preferred_element_type=jnp.float32)
                       + jnp.dot(xb, wmx_ref[...], preferred_element_type=jnp.float32)
                       + bm_ref[...])
        ls_ref[...] = (jnp.dot(h2, wlh_ref[...], preferred_element_type=jnp.float32)
                       + jnp.dot(xb, wlx_ref[...], preferred_element_type=jnp.float32)
                       + bl_ref[...])

    return pl.pallas_call(
        body,
        grid=grid,
        in_specs=[
            pl.BlockSpec((2, BN, HC), lambda i: (0, i, 0)),
            pl.BlockSpec((2, BN, HC), lambda i: (0, i, 0)),
            pl.BlockSpec((BN, x.shape[1]), lambda i: (i, 0)),
            pl.BlockSpec(Wmu_h.shape, lambda i: (0, 0)),
            pl.BlockSpec(Wmu_x.shape, lambda i: (0, 0)),
            pl.BlockSpec((1, Z), lambda i: (0, 0)),
            pl.BlockSpec(Wls_h.shape, lambda i: (0, 0)),
            pl.BlockSpec(Wls_x.shape, lambda i: (0, 0)),
            pl.BlockSpec((1, Z), lambda i: (0, 0)),
        ],
        out_specs=[
            pl.BlockSpec((BN, Z), lambda i: (i, 0)),
            pl.BlockSpec((BN, Z), lambda i: (i, 0)),
        ],
        out_shape=[
            jax.ShapeDtypeStruct((N_NODES, Z), jnp.float32),
            jax.ShapeDtypeStruct((N_NODES, Z), jnp.float32),
        ],
    )(y2, deg, x, Wmu_h, Wmu_x, bmu[None], Wls_h, Wls_x, bls[None])


def kernel(x, senders, receivers, W1, b1, W2, b2, Wmu, bmu, Wls, bls):
    E = senders.shape[0]
    n = x.shape[0]
    quantum = NSUB * EB * KCH * 2
    EP = ((E + quantum - 1) // quantum) * quantum
    pad = EP - E
    pad_n = jnp.full((pad,), n, jnp.int32)
    s_deg = jnp.concatenate([senders, pad_n]).reshape(-1, EB)
    r_pad = jnp.concatenate([receivers, pad_n]).reshape(-1, EB)
    s_gat = jnp.concatenate([senders, jnp.zeros((pad,), jnp.int32)]).reshape(-1, EB)

    ones_rows = jnp.ones((EB, HC), jnp.float32)
    zeros_rows = jnp.zeros((RPT, HC), jnp.float32)

    deg = _sc_degree(EP)(s_deg, r_pad, ones_rows, zeros_rows)
    z0, z1 = _tc_layer1(x, W1, b1, deg)
    y1 = _sc_message(EP)(z0, z1, s_gat, r_pad, zeros_rows)
    z20, z21 = _tc_layer2(y1, W2, b2, deg)
    y2 = _sc_message(EP)(z20, z21, s_gat, r_pad, zeros_rows)
    mu, logsig2 = _tc_heads(y2, deg, x, Wmu[:HID], Wmu[HID:], bmu,
                            Wls[:HID], Wls[HID:], bls)
    return (mu, logsig2)


# speedup vs baseline: 1.1346x; 1.0276x over previous
import functools

import jax
import jax.numpy as jnp
from jax import lax
from jax.experimental import pallas as pl
from jax.experimental.pallas import tpu as pltpu
from jax.experimental.pallas import tpu_sc as plsc

N_NODES = 100000
HID = 20
NSUB = 16
EB = 128
HC = 16
NP = N_NODES + HC
RPT = NP // NSUB
BN = 5000

_mesh = plsc.VectorSubcoreMesh(core_axis_name="c", subcore_axis_name="s")
_CP = pltpu.CompilerParams(use_tc_tiling_on_sc=False)


KCH = 23
NBUF = 8


def _sc_degree(EP):
    cpt = EP // EB // NSUB
    assert cpt % (2 * KCH) == 0
    half_groups = cpt // KCH // 2

    @functools.partial(
        pl.kernel,
        out_type=jax.ShapeDtypeStruct((2, NP, HC), jnp.float32),
        mesh=_mesh,
        compiler_params=_CP,
        scratch_types=[
            pltpu.VMEM((KCH, EB), jnp.int32),
            pltpu.VMEM((KCH, EB), jnp.int32),
            pltpu.VMEM((EB, HC), jnp.float32),
            pltpu.VMEM_SHARED((NP, HC), jnp.float32),
            pltpu.SemaphoreType.DMA,
            pltpu.SemaphoreType.DMA,
            pltpu.SemaphoreType.DMA,
        ],
    )
    def deg_kernel(s_hbm, r_hbm, ones_hbm, zeros_hbm, deg_out,
                   idx0, idx1, ones_v, acc, sem, isem0, isem1):
        c = lax.axis_index("c")
        s = lax.axis_index("s")
        r0 = s * RPT
        idx = (idx0, idx1)
        isem = (isem0, isem1)
        pltpu.sync_copy(zeros_hbm, acc.at[pl.ds(r0, RPT)])
        pltpu.sync_copy(ones_hbm, ones_v)
        plsc.subcore_barrier()

        def scan_edges(e_hbm):
            def scatter_block(p):
                ds = [pltpu.async_copy(ones_v, acc.at[idx[p].at[j]], sem,
                                       add=True) for j in range(KCH)]
                for d in ds:
                    d.wait()

            pltpu.async_copy(e_hbm.at[pl.ds(s * cpt, KCH)], idx[0], isem[0])

            def body(t, carry):
                base0 = s * cpt + 2 * t * KCH
                pltpu.make_async_copy(e_hbm.at[pl.ds(base0, KCH)],
                                      idx[0], isem[0]).wait()
                pltpu.async_copy(e_hbm.at[pl.ds(base0 + KCH, KCH)],
                                 idx[1], isem[1])
                scatter_block(0)
                pltpu.make_async_copy(e_hbm.at[pl.ds(base0 + KCH, KCH)],
                                      idx[1], isem[1]).wait()

                @pl.when(t + 1 < half_groups)
                def _():
                    pltpu.async_copy(e_hbm.at[pl.ds(base0 + 2 * KCH, KCH)],
                                     idx[0], isem[0])

                scatter_block(1)
                return carry
            lax.fori_loop(0, half_groups, body, 0)

        @pl.when(c == 0)
        def _():
            scan_edges(s_hbm)

        @pl.when(c == 1)
        def _():
            scan_edges(r_hbm)

        plsc.subcore_barrier()
        pltpu.sync_copy(acc.at[pl.ds(r0, RPT)], deg_out.at[c, pl.ds(r0, RPT)])

    return deg_kernel


def _sc_message(EP):
    cpt = EP // EB // NSUB
    assert cpt % (2 * KCH) == 0
    half_groups = cpt // KCH // 2

    @functools.partial(
        pl.kernel,
        out_type=jax.ShapeDtypeStruct((2, NP, HC), jnp.float32),
        mesh=_mesh,
        compiler_params=_CP,
        scratch_types=(
            [pltpu.VMEM((KCH, EB), jnp.int32) for _ in range(4)]
            + [pltpu.VMEM((EB, HC), jnp.float32) for _ in range(NBUF)]
            + [pltpu.VMEM_SHARED((NP, HC), jnp.float32)]
            + [pltpu.SemaphoreType.DMA for _ in range(2 * NBUF + 4)]
        ),
    )
    def msg_kernel(z0_hbm, z1_hbm, s_hbm, r_hbm, zeros_hbm, y_out, *refs):
        sidx = refs[0:2]
        ridx = refs[2:4]
        rows = refs[4:4 + NBUF]
        acc = refs[4 + NBUF]
        gsem = refs[5 + NBUF: 5 + 2 * NBUF]
        ssem = refs[5 + 2 * NBUF: 5 + 3 * NBUF]
        isem = refs[5 + 3 * NBUF:]
        c = lax.axis_index("c")
        s = lax.axis_index("s")
        r0 = s * RPT
        pltpu.sync_copy(zeros_hbm, acc.at[pl.ds(r0, RPT)])
        plsc.subcore_barrier()

        def fire_idx(base, p):
            pltpu.async_copy(s_hbm.at[pl.ds(base, KCH)], sidx[p], isem[p])
            pltpu.async_copy(r_hbm.at[pl.ds(base, KCH)], ridx[p], isem[2 + p])

        def wait_idx(base, p):
            pltpu.make_async_copy(s_hbm.at[pl.ds(base, KCH)], sidx[p],
                                  isem[p]).wait()
            pltpu.make_async_copy(r_hbm.at[pl.ds(base, KCH)], ridx[p],
                                  isem[2 + p]).wait()

        def process(z_hbm, sb, rb):
            dgat = [None] * NBUF
            dsc = [None] * NBUF
            for b in range(NBUF - 1):
                dgat[b] = pltpu.async_copy(z_hbm.at[sb.at[b]],
                                           rows[b], gsem[b])
            for j in range(KCH):
                b = j % NBUF
                dgat[b].wait()
                dsc[b] = pltpu.async_copy(rows[b], acc.at[rb.at[j]],
                                          ssem[b], add=True)
                nxt = j + NBUF - 1
                if nxt < KCH:
                    nb = nxt % NBUF
                    if dsc[nb] is not None:
                        dsc[nb].wait()
                    dgat[nb] = pltpu.async_copy(z_hbm.at[sb.at[nxt]],
                                                rows[nb], gsem[nb])
            for j in range(max(0, KCH - NBUF), KCH):
                dsc[j % NBUF].wait()

        def scan_edges(z_hbm):
            fire_idx(s * cpt, 0)

            def body(t, carry):
                base0 = s * cpt + 2 * t * KCH
                wait_idx(base0, 0)
                fire_idx(base0 + KCH, 1)
                process(z_hbm, sidx[0], ridx[0])
                wait_idx(base0 + KCH, 1)

                @pl.when(t + 1 < half_groups)
                def _():
                    fire_idx(base0 + 2 * KCH, 0)

                process(z_hbm, sidx[1], ridx[1])
                return carry
            lax.fori_loop(0, half_groups, body, 0)

        @pl.when(c == 0)
        def _():
            scan_edges(z0_hbm)

        @pl.when(c == 1)
        def _():
            scan_edges(z1_hbm)

        plsc.subcore_barrier()
        pltpu.sync_copy(acc.at[pl.ds(r0, RPT)], y_out.at[c, pl.ds(r0, RPT)])

    return msg_kernel


def _softmax_relu(h):
    e = jnp.exp(jnp.maximum(h, 0.0))
    return e / jnp.sum(e, axis=-1, keepdims=True)


def _split_cols(z):
    pad = jnp.zeros((z.shape[0], 2 * HC - HID), z.dtype)
    return z[:, :HC], jnp.concatenate([z[:, HC:], pad], axis=-1)


def _tc_layer1(x, W1, b1, deg):
    grid = (N_NODES // BN,)

    def body(x_ref, w_ref, b_ref, d_ref, o0_ref, o1_ref):
        h = jnp.dot(x_ref[...], w_ref[...], preferred_element_type=jnp.float32)
        z = _softmax_relu(h + b_ref[...])
        z = z * lax.rsqrt(jnp.maximum(d_ref[0][:, 0], 1.0))[:, None]
        z0, z1 = _split_cols(z)
        o0_ref[...] = z0
        o1_ref[...] = z1

    return pl.pallas_call(
        body,
        grid=grid,
        in_specs=[
            pl.BlockSpec((BN, x.shape[1]), lambda i: (i, 0)),
            pl.BlockSpec(W1.shape, lambda i: (0, 0)),
            pl.BlockSpec((1, HID), lambda i: (0, 0)),
            pl.BlockSpec((2, BN, HC), lambda i: (0, i, 0)),
        ],
        out_specs=[
            pl.BlockSpec((BN, HC), lambda i: (i, 0)),
            pl.BlockSpec((BN, HC), lambda i: (i, 0)),
        ],
        out_shape=[
            jax.ShapeDtypeStruct((N_NODES, HC), jnp.float32),
            jax.ShapeDtypeStruct((N_NODES, HC), jnp.float32),
        ],
    )(x, W1, b1[None], deg)


def _recombine(y_ref, d_ref):
    rd = d_ref[1][:, 0]
    h = jnp.concatenate([y_ref[0], y_ref[1][:, : HID - HC]], axis=-1)
    return h * lax.rsqrt(jnp.maximum(rd, 1.0))[:, None]


def _tc_layer2(y1, W2, b2, deg):
    grid = (N_NODES // BN,)

    def body(y_ref, w_ref, b_ref, d_ref, o0_ref, o1_ref):
        h1 = _recombine(y_ref, d_ref)
        h = jnp.dot(h1, w_ref[...], preferred_element_type=jnp.float32)
        z = _softmax_relu(h + b_ref[...])
        z = z * lax.rsqrt(jnp.maximum(d_ref[0][:, 0], 1.0))[:, None]
        z0, z1 = _split_cols(z)
        o0_ref[...] = z0
        o1_ref[...] = z1

    return pl.pallas_call(
        body,
        grid=grid,
        in_specs=[
            pl.BlockSpec((2, BN, HC), lambda i: (0, i, 0)),
            pl.BlockSpec(W2.shape, lambda i: (0, 0)),
            pl.BlockSpec((1, HID), lambda i: (0, 0)),
            pl.BlockSpec((2, BN, HC), lambda i: (0, i, 0)),
        ],
        out_specs=[
            pl.BlockSpec((BN, HC), lambda i: (i, 0)),
            pl.BlockSpec((BN, HC), lambda i: (i, 0)),
        ],
        out_shape=[
            jax.ShapeDtypeStruct((N_NODES, HC), jnp.float32),
            jax.ShapeDtypeStruct((N_NODES, HC), jnp.float32),
        ],
    )(y1, W2, b2[None], deg)


def _tc_heads(y2, deg, x, Wmu_h, Wmu_x, bmu, Wls_h, Wls_x, bls):
    grid = (N_NODES // BN,)
    Z = Wmu_h.shape[1]

    def body(y_ref, d_ref, x_ref, wmh_ref, wmx_ref, bm_ref,
             wlh_ref, wlx_ref, bl_ref, mu_ref, ls_ref):
        h2 = _recombine(y_ref, d_ref)
        xb = x_ref[...]
        mu_ref[...] = (jnp.dot(h2, wmh_ref[...], preferred_element_type=jnp.float32)
                       + jnp.dot(xb, wmx_ref[...], preferred_element_type=jnp.float32)
                       + bm_ref[...])
        ls_ref[...] = (jnp.dot(h2, wlh_ref[...], preferred_element_type=jnp.float32)
                       + jnp.dot(xb, wlx_ref[...], preferred_element_type=jnp.float32)
                       + bl_ref[...])

    return pl.pallas_call(
        body,
        grid=grid,
        in_specs=[
            pl.BlockSpec((2, BN, HC), lambda i: (0, i, 0)),
            pl.BlockSpec((2, BN, HC), lambda i: (0, i, 0)),
            pl.BlockSpec((BN, x.shape[1]), lambda i: (i, 0)),
            pl.BlockSpec(Wmu_h.shape, lambda i: (0, 0)),
            pl.BlockSpec(Wmu_x.shape, lambda i: (0, 0)),
            pl.BlockSpec((1, Z), lambda i: (0, 0)),
            pl.BlockSpec(Wls_h.shape, lambda i: (0, 0)),
            pl.BlockSpec(Wls_x.shape, lambda i: (0, 0)),
            pl.BlockSpec((1, Z), lambda i: (0, 0)),
        ],
        out_specs=[
            pl.BlockSpec((BN, Z), lambda i: (i, 0)),
            pl.BlockSpec((BN, Z), lambda i: (i, 0)),
        ],
        out_shape=[
            jax.ShapeDtypeStruct((N_NODES, Z), jnp.float32),
            jax.ShapeDtypeStruct((N_NODES, Z), jnp.float32),
        ],
    )(y2, deg, x, Wmu_h, Wmu_x, bmu[None], Wls_h, Wls_x, bls[None])


def kernel(x, senders, receivers, W1, b1, W2, b2, Wmu, bmu, Wls, bls):
    E = senders.shape[0]
    n = x.shape[0]
    quantum = NSUB * EB * KCH * 2
    EP = ((E + quantum - 1) // quantum) * quantum
    pad = EP - E
    pad_n = jnp.full((pad,), n, jnp.int32)
    s_deg = jnp.concatenate([senders, pad_n]).reshape(-1, EB)
    r_pad = jnp.concatenate([receivers, pad_n]).reshape(-1, EB)
    s_gat = jnp.concatenate([senders, jnp.zeros((pad,), jnp.int32)]).reshape(-1, EB)

    ones_rows = jnp.ones((EB, HC), jnp.float32)
    zeros_rows = jnp.zeros((RPT, HC), jnp.float32)

    deg = _sc_degree(EP)(s_deg, r_pad, ones_rows, zeros_rows)
    z0, z1 = _tc_layer1(x, W1, b1, deg)
    y1 = _sc_message(EP)(z0, z1, s_gat, r_pad, zeros_rows)
    z20, z21 = _tc_layer2(y1, W2, b2, deg)
    y2 = _sc_message(EP)(z20, z21, s_gat, r_pad, zeros_rows)
    mu, logsig2 = _tc_heads(y2, deg, x, Wmu[:HID], Wmu[HID:], bmu,
                            Wls[:HID], Wls[HID:], bls)
    return (mu, logsig2)
